# baseline (speedup 1.0000x reference)
import functools
import math

import jax
import jax.numpy as jnp
from jax.experimental import pallas as pl
from jax.experimental.pallas import tpu as pltpu

VOCAB = 8192
DIM = 1024
HEADS = 16
LAYERS = 2
FF = 4 * DIM
S = 2048
DH = DIM // HEADS

EPS = 1e-6
FC = 1024
VC = 1024


def _rms(x, w):
    return x * jax.lax.rsqrt(jnp.mean(x * x, axis=-1, keepdims=True) + EPS) * w


def _gather_body(ids_ref, emb_ref, o_ref):
    o_ref[...] = emb_ref[...]


def _embed_gather(text_ids, embed):
    grid_spec = pltpu.PrefetchScalarGridSpec(
        num_scalar_prefetch=1,
        grid=(S,),
        in_specs=[pl.BlockSpec((1, 1, DIM), lambda i, ids: (ids[i], 0, 0))],
        out_specs=pl.BlockSpec((1, 1, DIM), lambda i, ids: (i, 0, 0)),
    )
    return pl.pallas_call(
        _gather_body,
        grid_spec=grid_spec,
        out_shape=jax.ShapeDtypeStruct((S, 1, DIM), jnp.float32),
    )(text_ids.reshape(S), embed.reshape(VOCAB, 1, DIM)).reshape(S, DIM)


HG = 2
GD = HG * DH
QC = 512
NQC = S // QC


def _attn_body(x_ref, n1_ref, wq_ref, wk_ref, wv_ref, wo_ref, o_ref,
               xn_ref, kg_ref, vg_ref):
    i = pl.program_id(0)
    j = pl.program_id(1)

    @pl.when(jnp.logical_and(i == 0, j == 0))
    def _():
        xn_ref[...] = _rms(x_ref[...], n1_ref[...]).astype(jnp.bfloat16)

    @pl.when(j == 0)
    def _():
        xn = xn_ref[...]
        kg_ref[...] = jnp.dot(
            xn, wk_ref[...], preferred_element_type=jnp.float32
        ).astype(jnp.bfloat16)
        vg_ref[...] = jnp.dot(
            xn, wv_ref[...], preferred_element_type=jnp.float32
        ).astype(jnp.bfloat16)

    rows = pl.ds(j * QC, QC)
    qg = jnp.dot(
        xn_ref[rows, :], wq_ref[...], preferred_element_type=jnp.float32
    ).astype(jnp.bfloat16)
    og_parts = []
    for h in range(HG):
        cols = slice(h * DH, (h + 1) * DH)
        q = qg[:, cols]
        k = kg_ref[:, cols]
        v = vg_ref[:, cols]
        s = jax.lax.dot_general(
            q, k, (((1,), (1,)), ((), ())), preferred_element_type=jnp.float32
        ) * (1.0 / math.sqrt(DH))
        s = s - jnp.max(s, axis=-1, keepdims=True)
        p = jnp.exp(s)
        p = (p / jnp.sum(p, axis=-1, keepdims=True)).astype(jnp.bfloat16)
        og_parts.append(jnp.dot(p, v, preferred_element_type=jnp.float32))
    og = jnp.concatenate(og_parts, axis=1).astype(jnp.bfloat16)
    contrib = jnp.dot(og, wo_ref[...], preferred_element_type=jnp.float32)

    @pl.when(i == 0)
    def _():
        o_ref[rows, :] = x_ref[rows, :] + contrib

    @pl.when(i > 0)
    def _():
        o_ref[rows, :] += contrib


def _attn_block(x, n1, wq, wk, wv, wo):
    return pl.pallas_call(
        _attn_body,
        grid=(HEADS // HG, NQC),
        in_specs=[
            pl.BlockSpec((S, DIM), lambda i, j: (0, 0)),
            pl.BlockSpec((1, DIM), lambda i, j: (0, 0)),
            pl.BlockSpec((DIM, GD), lambda i, j: (0, i)),
            pl.BlockSpec((DIM, GD), lambda i, j: (0, i)),
            pl.BlockSpec((DIM, GD), lambda i, j: (0, i)),
            pl.BlockSpec((GD, DIM), lambda i, j: (i, 0)),
        ],
        out_specs=pl.BlockSpec((S, DIM), lambda i, j: (0, 0)),
        out_shape=jax.ShapeDtypeStruct((S, DIM), jnp.float32),
        scratch_shapes=[
            pltpu.VMEM((S, DIM), jnp.bfloat16),
            pltpu.VMEM((S, GD), jnp.bfloat16),
            pltpu.VMEM((S, GD), jnp.bfloat16),
        ],
        compiler_params=pltpu.CompilerParams(
            dimension_semantics=("arbitrary", "arbitrary"),
        ),
    )(x, n1, wq.astype(jnp.bfloat16), wk.astype(jnp.bfloat16),
      wv.astype(jnp.bfloat16), wo.astype(jnp.bfloat16))


def _ffn_body(x_ref, n2_ref, w1_ref, b1_ref, w2_ref, b2_ref, o_ref, xn_ref):
    j = pl.program_id(0)

    @pl.when(j == 0)
    def _():
        xn_ref[...] = _rms(x_ref[...], n2_ref[...]).astype(jnp.bfloat16)

    h = jnp.dot(xn_ref[...], w1_ref[...], preferred_element_type=jnp.float32)
    h = h + b1_ref[...]
    h = (h * jax.nn.sigmoid(h)).astype(jnp.bfloat16)
    c = jnp.dot(h, w2_ref[...], preferred_element_type=jnp.float32)

    @pl.when(j == 0)
    def _():
        o_ref[...] = x_ref[...] + b2_ref[...] + c

    @pl.when(j > 0)
    def _():
        o_ref[...] += c


def _ffn_block(x, n2, w1, b1, w2, b2):
    return pl.pallas_call(
        _ffn_body,
        grid=(FF // FC,),
        in_specs=[
            pl.BlockSpec((S, DIM), lambda j: (0, 0)),
            pl.BlockSpec((1, DIM), lambda j: (0, 0)),
            pl.BlockSpec((DIM, FC), lambda j: (0, j)),
            pl.BlockSpec((1, FC), lambda j: (0, j)),
            pl.BlockSpec((FC, DIM), lambda j: (j, 0)),
            pl.BlockSpec((1, DIM), lambda j: (0, 0)),
        ],
        out_specs=pl.BlockSpec((S, DIM), lambda j: (0, 0)),
        out_shape=jax.ShapeDtypeStruct((S, DIM), jnp.float32),
        scratch_shapes=[pltpu.VMEM((S, DIM), jnp.bfloat16)],
        compiler_params=pltpu.CompilerParams(
            dimension_semantics=("arbitrary",),
        ),
    )(x, n2, w1.astype(jnp.bfloat16), b1.reshape(1, FF),
      w2.astype(jnp.bfloat16), b2.reshape(1, DIM))


def _head_body(x_ref, fw_ref, w_ref, b_ref, o_ref, xn_ref):
    j = pl.program_id(0)

    @pl.when(j == 0)
    def _():
        xn_ref[...] = _rms(x_ref[...], fw_ref[...]).astype(jnp.bfloat16)

    o_ref[...] = (
        jnp.dot(xn_ref[...], w_ref[...], preferred_element_type=jnp.float32)
        + b_ref[...]
    )


def _head_block(x, fw, w, b):
    return pl.pallas_call(
        _head_body,
        grid=(VOCAB // VC,),
        in_specs=[
            pl.BlockSpec((S, DIM), lambda j: (0, 0)),
            pl.BlockSpec((1, DIM), lambda j: (0, 0)),
            pl.BlockSpec((DIM, VC), lambda j: (0, j)),
            pl.BlockSpec((1, VC), lambda j: (0, j)),
        ],
        out_specs=pl.BlockSpec((S, VC), lambda j: (0, j)),
        out_shape=jax.ShapeDtypeStruct((S, VOCAB), jnp.float32),
        scratch_shapes=[pltpu.VMEM((S, DIM), jnp.bfloat16)],
        compiler_params=pltpu.CompilerParams(
            dimension_semantics=("arbitrary",),
        ),
    )(x, fw.reshape(1, DIM), w.astype(jnp.bfloat16), b.reshape(1, VOCAB))


@jax.jit
def kernel(text_ids, embed, norm1_w, norm2_w, Wq, Wk, Wv, Wo, W1, b1, W2, b2,
           final_w, head_W, head_b):
    x = _embed_gather(text_ids, embed)
    for l in range(LAYERS):
        x = _attn_block(x, norm1_w[l].reshape(1, DIM), Wq[l], Wk[l], Wv[l], Wo[l])
        x = _ffn_block(x, norm2_w[l].reshape(1, DIM), W1[l], b1[l], W2[l], b2[l])
    logits = _head_block(x, final_w, head_W, head_b)
    return logits.reshape(1, S, VOCAB)


# speedup vs baseline: 2.3000x; 2.3000x over previous
import functools
import math

import jax
import jax.numpy as jnp
from jax.experimental import pallas as pl
from jax.experimental.pallas import tpu as pltpu

VOCAB = 8192
DIM = 1024
HEADS = 16
LAYERS = 2
FF = 4 * DIM
S = 2048
DH = DIM // HEADS

EPS = 1e-6
FC = 1024
VC = 1024


def _rms(x, w):
    return x * jax.lax.rsqrt(jnp.mean(x * x, axis=-1, keepdims=True) + EPS) * w


def _gather_body(ids_ref, emb_ref, o_ref):
    o_ref[...] = emb_ref[...]


def _embed_gather(text_ids, embed):
    grid_spec = pltpu.PrefetchScalarGridSpec(
        num_scalar_prefetch=1,
        grid=(S,),
        in_specs=[pl.BlockSpec((1, 1, DIM), lambda i, ids: (ids[i], 0, 0))],
        out_specs=pl.BlockSpec((1, 1, DIM), lambda i, ids: (i, 0, 0)),
    )
    return pl.pallas_call(
        _gather_body,
        grid_spec=grid_spec,
        out_shape=jax.ShapeDtypeStruct((S, 1, DIM), jnp.float32),
    )(text_ids.reshape(S), embed.reshape(VOCAB, 1, DIM)).reshape(S, DIM)


HG = 2
GD = HG * DH
QC = 512
NQC = S // QC


def _attn_body(x_ref, n1_ref, wq_ref, wk_ref, wv_ref, wo_ref, o_ref,
               xn_ref, kg_ref, vg_ref):
    i = pl.program_id(0)
    j = pl.program_id(1)

    @pl.when(jnp.logical_and(i == 0, j == 0))
    def _():
        xn_ref[...] = _rms(x_ref[...], n1_ref[...]).astype(jnp.bfloat16)

    @pl.when(j == 0)
    def _():
        xn = xn_ref[...]
        kg_ref[...] = jnp.dot(
            xn, wk_ref[...], preferred_element_type=jnp.float32
        ).astype(jnp.bfloat16)
        vg_ref[...] = jnp.dot(
            xn, wv_ref[...], preferred_element_type=jnp.float32
        ).astype(jnp.bfloat16)

    rows = pl.ds(j * QC, QC)
    qg = jnp.dot(
        xn_ref[rows, :], wq_ref[...], preferred_element_type=jnp.float32
    ).astype(jnp.bfloat16)
    og_parts = []
    for h in range(HG):
        cols = slice(h * DH, (h + 1) * DH)
        q = qg[:, cols]
        k = kg_ref[:, cols]
        v = vg_ref[:, cols]
        s = jax.lax.dot_general(
            q, k, (((1,), (1,)), ((), ())), preferred_element_type=jnp.float32
        ) * (1.0 / math.sqrt(DH))
        s = s - jnp.max(s, axis=-1, keepdims=True)
        p = jnp.exp(s)
        p = (p / jnp.sum(p, axis=-1, keepdims=True)).astype(jnp.bfloat16)
        og_parts.append(jnp.dot(p, v, preferred_element_type=jnp.float32))
    og = jnp.concatenate(og_parts, axis=1).astype(jnp.bfloat16)
    contrib = jnp.dot(og, wo_ref[...], preferred_element_type=jnp.float32)

    @pl.when(i == 0)
    def _():
        o_ref[rows, :] = x_ref[rows, :] + contrib

    @pl.when(i > 0)
    def _():
        o_ref[rows, :] += contrib


def _attn_block(x, n1, wq, wk, wv, wo):
    return pl.pallas_call(
        _attn_body,
        grid=(HEADS // HG, NQC),
        in_specs=[
            pl.BlockSpec((S, DIM), lambda i, j: (0, 0)),
            pl.BlockSpec((1, DIM), lambda i, j: (0, 0)),
            pl.BlockSpec((DIM, GD), lambda i, j: (0, i)),
            pl.BlockSpec((DIM, GD), lambda i, j: (0, i)),
            pl.BlockSpec((DIM, GD), lambda i, j: (0, i)),
            pl.BlockSpec((GD, DIM), lambda i, j: (i, 0)),
        ],
        out_specs=pl.BlockSpec((S, DIM), lambda i, j: (0, 0)),
        out_shape=jax.ShapeDtypeStruct((S, DIM), jnp.float32),
        scratch_shapes=[
            pltpu.VMEM((S, DIM), jnp.bfloat16),
            pltpu.VMEM((S, GD), jnp.bfloat16),
            pltpu.VMEM((S, GD), jnp.bfloat16),
        ],
        compiler_params=pltpu.CompilerParams(
            dimension_semantics=("arbitrary", "arbitrary"),
        ),
    )(x, n1, wq.astype(jnp.bfloat16), wk.astype(jnp.bfloat16),
      wv.astype(jnp.bfloat16), wo.astype(jnp.bfloat16))


def _ffn_body(x_ref, n2_ref, w1_ref, b1_ref, w2_ref, b2_ref, o_ref, xn_ref):
    j = pl.program_id(0)

    @pl.when(j == 0)
    def _():
        xn_ref[...] = _rms(x_ref[...], n2_ref[...]).astype(jnp.bfloat16)

    h = jnp.dot(xn_ref[...], w1_ref[...], preferred_element_type=jnp.float32)
    h = h + b1_ref[...]
    h = (h * jax.nn.sigmoid(h)).astype(jnp.bfloat16)
    c = jnp.dot(h, w2_ref[...], preferred_element_type=jnp.float32)

    @pl.when(j == 0)
    def _():
        o_ref[...] = x_ref[...] + b2_ref[...] + c

    @pl.when(j > 0)
    def _():
        o_ref[...] += c


def _ffn_block(x, n2, w1, b1, w2, b2):
    return pl.pallas_call(
        _ffn_body,
        grid=(FF // FC,),
        in_specs=[
            pl.BlockSpec((S, DIM), lambda j: (0, 0)),
            pl.BlockSpec((1, DIM), lambda j: (0, 0)),
            pl.BlockSpec((DIM, FC), lambda j: (0, j)),
            pl.BlockSpec((1, FC), lambda j: (0, j)),
            pl.BlockSpec((FC, DIM), lambda j: (j, 0)),
            pl.BlockSpec((1, DIM), lambda j: (0, 0)),
        ],
        out_specs=pl.BlockSpec((S, DIM), lambda j: (0, 0)),
        out_shape=jax.ShapeDtypeStruct((S, DIM), jnp.float32),
        scratch_shapes=[pltpu.VMEM((S, DIM), jnp.bfloat16)],
        compiler_params=pltpu.CompilerParams(
            dimension_semantics=("arbitrary",),
        ),
    )(x, n2, w1.astype(jnp.bfloat16), b1.reshape(1, FF),
      w2.astype(jnp.bfloat16), b2.reshape(1, DIM))


def _head_body(x_ref, fw_ref, w_ref, b_ref, o_ref, xn_ref):
    j = pl.program_id(0)

    @pl.when(j == 0)
    def _():
        xn_ref[...] = _rms(x_ref[...], fw_ref[...]).astype(jnp.bfloat16)

    o_ref[...] = (
        jnp.dot(xn_ref[...], w_ref[...], preferred_element_type=jnp.float32)
        + b_ref[...]
    )


def _head_block(x, fw, w, b):
    return pl.pallas_call(
        _head_body,
        grid=(VOCAB // VC,),
        in_specs=[
            pl.BlockSpec((S, DIM), lambda j: (0, 0)),
            pl.BlockSpec((1, DIM), lambda j: (0, 0)),
            pl.BlockSpec((DIM, VC), lambda j: (0, j)),
            pl.BlockSpec((1, VC), lambda j: (0, j)),
        ],
        out_specs=pl.BlockSpec((S, VC), lambda j: (0, j)),
        out_shape=jax.ShapeDtypeStruct((S, VOCAB), jnp.float32),
        scratch_shapes=[pltpu.VMEM((S, DIM), jnp.bfloat16)],
        compiler_params=pltpu.CompilerParams(
            dimension_semantics=("arbitrary",),
        ),
    )(x, fw.reshape(1, DIM), w.astype(jnp.bfloat16), b.reshape(1, VOCAB))


@jax.jit
def kernel(text_ids, embed, norm1_w, norm2_w, Wq, Wk, Wv, Wo, W1, b1, W2, b2,
           final_w, head_W, head_b):
    x = jnp.take(embed, text_ids.reshape(S), axis=0)
    for l in range(LAYERS):
        x = _attn_block(x, norm1_w[l].reshape(1, DIM), Wq[l], Wk[l], Wv[l], Wo[l])
        x = _ffn_block(x, norm2_w[l].reshape(1, DIM), W1[l], b1[l], W2[l], b2[l])
    logits = _head_block(x, final_w, head_W, head_b)
    return logits.reshape(1, S, VOCAB)
